# Initial kernel scaffold; baseline (speedup 1.0000x reference)
import jax
import jax.numpy as jnp
from jax.experimental import pallas as pl


def kernel(users, pos, neg, reg, user_emb, item_emb, adj_rows, adj_cols, adj_vals):
    raise NotImplementedError("write your pallas kernel here")


# speedup vs baseline: 7.5004x; 7.5004x over previous
import functools
import math

import jax
import jax.numpy as jnp
from jax import lax
from jax.experimental import pallas as pl
from jax.experimental.pallas import tpu as pltpu
from jax.experimental.pallas import tpu_sc as plsc

DIM = 32
LANES = 16
KB = 128
NS = 16
NC = 2


def _pad1d(x, n, fill):
    if x.shape[0] == n:
        return x
    return jnp.concatenate([x, jnp.full((n - x.shape[0],), fill, x.dtype)])


@functools.lru_cache(maxsize=None)
def _make_propagate(nn, nu, nbh):
    half_pad = nbh * KB
    ni = nn - nu
    g = math.gcd(nu, ni)
    rz = max(d for d in range(8, KB + 1, 8) if g % d == 0)
    rc = max(d for d in range(8, 2049, 8) if g % d == 0)
    mesh = plsc.VectorSubcoreMesh(core_axis_name="c", subcore_axis_name="s")

    @functools.partial(
        pl.kernel,
        mesh=mesh,
        compiler_params=pltpu.CompilerParams(use_tc_tiling_on_sc=False),
        out_type=jax.ShapeDtypeStruct((nn, DIM), jnp.float32),
        scratch_types=[
            pltpu.VMEM((KB,), jnp.int32),
            pltpu.VMEM((KB,), jnp.int32),
            pltpu.VMEM((KB,), jnp.float32),
            pltpu.VMEM((KB, DIM), jnp.float32),
            pltpu.VMEM_SHARED((nn, DIM), jnp.float32),
            pltpu.SemaphoreType.DMA,
        ],
    )
    def prop(emb, rows, cols, vals, out, colsb, rowsb, valsb, gbuf, acc, sem):
        c = lax.axis_index("c")
        s = lax.axis_index("s")
        zero16 = jnp.zeros((LANES,), jnp.float32)

        def zb(i, carry):
            gbuf[i, pl.ds(0, LANES)] = zero16
            gbuf[i, pl.ds(LANES, LANES)] = zero16
            return carry

        lax.fori_loop(0, KB, zb, 0)

        rbase = jnp.where(c == 0, 0, nu)
        nzb = jnp.where(c == 0, nu // rz, ni // rz)
        nzb_s = (nzb - 1 - s) // NS + 1

        def zacc(i, carry):
            off = rbase + (s + i * NS) * rz
            pltpu.sync_copy(gbuf.at[pl.ds(0, rz)], acc.at[pl.ds(off, rz)])
            return carry

        lax.fori_loop(0, nzb_s, zacc, 0)
        plsc.subcore_barrier()

        nblk = (nbh - 1 - s) // NS + 1

        def eb(i, carry):
            b = s + i * NS
            base = c * half_pad + b * KB
            pltpu.sync_copy(cols.at[pl.ds(base, KB)], colsb)
            pltpu.sync_copy(rows.at[pl.ds(base, KB)], rowsb)
            pltpu.sync_copy(vals.at[pl.ds(base, KB)], valsb)
            pltpu.async_copy(emb.at[colsb], gbuf, sem).wait()

            def scale(k, carry2):
                vv = valsb[pl.ds(k * LANES, LANES)]
                for j in range(LANES):
                    e = k * LANES + j
                    v = vv[j]
                    gbuf[e, pl.ds(0, LANES)] = gbuf[e, pl.ds(0, LANES)] * v
                    gbuf[e, pl.ds(LANES, LANES)] = gbuf[e, pl.ds(LANES, LANES)] * v
                return carry2

            lax.fori_loop(0, KB // LANES, scale, 0)
            pltpu.sync_copy(gbuf, acc.at[rowsb], add=True)
            return carry

        lax.fori_loop(0, nblk, eb, 0)
        plsc.subcore_barrier()

        nwb = jnp.where(c == 0, nu // rc, ni // rc)
        nwb_s = (nwb - 1 - s) // NS + 1

        def wb(i, carry):
            off = rbase + (s + i * NS) * rc
            pltpu.sync_copy(acc.at[pl.ds(off, rc)], out.at[pl.ds(off, rc)])
            return carry

        lax.fori_loop(0, nwb_s, wb, 0)

    return prop


@functools.lru_cache(maxsize=None)
def _make_gather(nn, nb2):
    nw = NC * NS
    mesh = plsc.VectorSubcoreMesh(core_axis_name="c", subcore_axis_name="s")

    @functools.partial(
        pl.kernel,
        mesh=mesh,
        compiler_params=pltpu.CompilerParams(use_tc_tiling_on_sc=False),
        out_type=[
            jax.ShapeDtypeStruct((nb2 * KB, DIM), jnp.float32),
            jax.ShapeDtypeStruct((nb2 * KB, DIM), jnp.float32),
        ],
        scratch_types=[
            pltpu.VMEM((KB,), jnp.int32),
            pltpu.VMEM((KB, DIM), jnp.float32),
            pltpu.VMEM((KB, DIM), jnp.float32),
            pltpu.SemaphoreType.DMA,
        ],
    )
    def gath(e0, e1, e2, e3, idx, osum, o0, ib, buf0, bufa, sem):
        c = lax.axis_index("c")
        s = lax.axis_index("s")
        w = s * NC + c
        nblk = (nb2 - 1 - w) // nw + 1

        def blk(i, carry):
            b = w + i * nw
            base = b * KB
            pltpu.sync_copy(idx.at[pl.ds(base, KB)], ib)
            pltpu.async_copy(e0.at[ib], buf0, sem).wait()
            pltpu.sync_copy(buf0, o0.at[pl.ds(base, KB)])
            for tab in (e1, e2, e3):
                pltpu.async_copy(tab.at[ib], bufa, sem).wait()

                def add_(k, carry2):
                    buf0[k, pl.ds(0, LANES)] = (
                        buf0[k, pl.ds(0, LANES)] + bufa[k, pl.ds(0, LANES)]
                    )
                    buf0[k, pl.ds(LANES, LANES)] = (
                        buf0[k, pl.ds(LANES, LANES)] + bufa[k, pl.ds(LANES, LANES)]
                    )
                    return carry2

                lax.fori_loop(0, KB, add_, 0)
            pltpu.sync_copy(buf0, osum.at[pl.ds(base, KB)])
            return carry

        lax.fori_loop(0, nblk, blk, 0)

    return gath


def _loss_body(us_ref, ps_ref, ns_ref, u0_ref, p0_ref, n0_ref, reg_ref, out_ref, *, total):
    step = pl.program_id(0)

    @pl.when(step == 0)
    def _init():
        out_ref[0, 0] = 0.0

    us = us_ref[...] * 0.25
    ps = ps_ref[...] * 0.25
    ns = ns_ref[...] * 0.25
    pos_s = jnp.sum(us * ps, axis=1)
    neg_s = jnp.sum(us * ns, axis=1)
    x = pos_s - neg_s
    logsig = jnp.minimum(x, 0.0) - jnp.log(1.0 + jnp.exp(-jnp.abs(x)))
    r = reg_ref[0, 0]
    sq = (
        jnp.sum(u0_ref[...] * u0_ref[...])
        + jnp.sum(p0_ref[...] * p0_ref[...])
        + jnp.sum(n0_ref[...] * n0_ref[...])
    )
    part = -jnp.sum(logsig) / total + jnp.where(r > 0, r * sq / total, 0.0)
    out_ref[0, 0] += part


def _loss_tc(usum, psum, nsum, u0, p0, n0, regf):
    b = usum.shape[0]
    blk = 1024 if b % 1024 == 0 else b
    grid = b // blk
    row_spec = pl.BlockSpec((blk, DIM), lambda i: (i, 0))
    out = pl.pallas_call(
        functools.partial(_loss_body, total=float(b)),
        grid=(grid,),
        in_specs=[row_spec] * 6 + [pl.BlockSpec(memory_space=pltpu.SMEM)],
        out_specs=pl.BlockSpec(memory_space=pltpu.SMEM),
        out_shape=jax.ShapeDtypeStruct((1, 1), jnp.float32),
    )(usum, psum, nsum, u0, p0, n0, regf)
    return out[0, 0]


def kernel(users, pos, neg, reg, user_emb, item_emb, adj_rows, adj_cols, adj_vals):
    nu = user_emb.shape[0]
    ni = item_emb.shape[0]
    nn = nu + ni
    nnz = adj_rows.shape[0]
    half = nnz // 2
    nbh = -(-half // KB)
    half_pad = nbh * KB

    rows_p = jnp.concatenate(
        [_pad1d(adj_rows[:half], half_pad, 0), _pad1d(adj_rows[half:], half_pad, 0)]
    )
    cols_p = jnp.concatenate(
        [_pad1d(adj_cols[:half], half_pad, 0), _pad1d(adj_cols[half:], half_pad, 0)]
    )
    vals_p = jnp.concatenate(
        [_pad1d(adj_vals[:half], half_pad, 0.0), _pad1d(adj_vals[half:], half_pad, 0.0)]
    )

    e0 = jnp.concatenate([user_emb, item_emb], axis=0)
    prop = _make_propagate(nn, nu, nbh)
    e1 = prop(e0, rows_p, cols_p, vals_p)
    e2 = prop(e1, rows_p, cols_p, vals_p)
    e3 = prop(e2, rows_p, cols_p, vals_p)

    b = users.shape[0]
    idx = jnp.concatenate(
        [users.astype(jnp.int32), pos.astype(jnp.int32) + nu, neg.astype(jnp.int32) + nu]
    )
    tot = 3 * b
    nb2 = -(-tot // KB)
    idx_p = _pad1d(idx, nb2 * KB, 0)
    gath = _make_gather(nn, nb2)
    gsum, g0 = gath(e0, e1, e2, e3, idx_p)

    regf = jnp.asarray(reg, jnp.float32).reshape(1, 1)
    return _loss_tc(
        gsum[:b], gsum[b : 2 * b], gsum[2 * b : tot],
        g0[:b], g0[b : 2 * b], g0[2 * b : tot],
        regf,
    )


# speedup vs baseline: 23.4281x; 3.1236x over previous
import functools
import math

import jax
import jax.numpy as jnp
from jax import lax
from jax.experimental import pallas as pl
from jax.experimental.pallas import tpu as pltpu
from jax.experimental.pallas import tpu_sc as plsc

DIM = 32
LANES = 16
KB = 128
NS = 16
NC = 2


def _pad1d(x, n, fill):
    if x.shape[0] == n:
        return x
    return jnp.concatenate([x, jnp.full((n - x.shape[0],), fill, x.dtype)])


SB = 896


@functools.lru_cache(maxsize=None)
def _make_propagate(nn, nu, nsb):
    per_tile = nsb * SB
    half_pad = NS * per_tile
    ni = nn - nu
    g = math.gcd(nu, ni)
    rz = max(d for d in range(8, KB + 1, 8) if g % d == 0)
    rc = max(d for d in range(8, 2049, 8) if g % d == 0)
    mesh = plsc.VectorSubcoreMesh(core_axis_name="c", subcore_axis_name="s")

    @functools.partial(
        pl.kernel,
        mesh=mesh,
        compiler_params=pltpu.CompilerParams(use_tc_tiling_on_sc=False),
        out_type=jax.ShapeDtypeStruct((nn, DIM), jnp.float32),
        scratch_types=[
            pltpu.VMEM((2, SB), jnp.int32),
            pltpu.VMEM((2, SB), jnp.int32),
            pltpu.VMEM((2, SB), jnp.float32),
            pltpu.VMEM((SB, DIM), jnp.float32),
            pltpu.VMEM((SB, DIM), jnp.float32),
            pltpu.VMEM_SHARED((max(nu, nn - nu), DIM), jnp.float32),
            pltpu.SemaphoreType.DMA,
            pltpu.SemaphoreType.DMA,
            pltpu.SemaphoreType.DMA,
        ],
    )
    def prop(emb, rows, cols, vals, out, colsb, rowsb, valsb, gb0, gb1, acc,
             semi, semg0, semg1):
        c = lax.axis_index("c")
        s = lax.axis_index("s")
        gbs = (gb0, gb1)
        semg = (semg0, semg1)
        zero16 = jnp.zeros((LANES,), jnp.float32)

        def zb(i, carry):
            gb0[i, pl.ds(0, LANES)] = zero16
            gb0[i, pl.ds(LANES, LANES)] = zero16
            return carry

        lax.fori_loop(0, KB, zb, 0)

        rbase = jnp.where(c == 0, 0, nu)
        nrows = jnp.where(c == 0, nu, ni)
        nzb_s = (nrows // rz - 1 - s) // NS + 1

        def zacc(i, carry):
            off = (s + i * NS) * rz
            pltpu.sync_copy(gb0.at[pl.ds(0, rz)], acc.at[pl.ds(off, rz)])
            return carry

        lax.fori_loop(0, nzb_s, zacc, 0)
        plsc.subcore_barrier()

        tile_e0 = (c * NS + s) * per_tile

        def fire_idx(i, p):
            base = tile_e0 + i * SB
            pltpu.async_copy(cols.at[pl.ds(base, SB)], colsb.at[p], semi)
            pltpu.async_copy(rows.at[pl.ds(base, SB)], rowsb.at[p], semi)
            pltpu.async_copy(vals.at[pl.ds(base, SB)], valsb.at[p], semi)

        def wait_idx(i, p):
            base = tile_e0 + i * SB
            pltpu.make_async_copy(cols.at[pl.ds(base, SB)], colsb.at[p], semi).wait()
            pltpu.make_async_copy(rows.at[pl.ds(base, SB)], rowsb.at[p], semi).wait()
            pltpu.make_async_copy(vals.at[pl.ds(base, SB)], valsb.at[p], semi).wait()

        def sub_rows(p):
            def sr(k, carry):
                sl = pl.ds(k * LANES, LANES)
                rowsb[p, sl] = rowsb[p, sl] - rbase
                return carry

            lax.fori_loop(0, SB // LANES, sr, 0)

        def fire_gathers(p):
            pltpu.async_copy(emb.at[colsb.at[p]], gbs[p], semg[p])

        def wait_gathers(p):
            pltpu.make_async_copy(emb.at[colsb.at[p]], gbs[p], semg[p]).wait()

        def do_scatter(p):
            pltpu.sync_copy(gbs[p], acc.at[rowsb.at[p]], add=True)

        def scale(p):
            def sc16(k, carry):
                vv = valsb[p, pl.ds(k * LANES, LANES)]
                for j in range(LANES):
                    e = k * LANES + j
                    v = vv[j]
                    gbs[p][e, pl.ds(0, LANES)] = gbs[p][e, pl.ds(0, LANES)] * v
                    gbs[p][e, pl.ds(LANES, LANES)] = (
                        gbs[p][e, pl.ds(LANES, LANES)] * v
                    )
                return carry

            lax.fori_loop(0, SB // LANES, sc16, 0)

        fire_idx(0, 0)
        wait_idx(0, 0)
        sub_rows(0)
        fire_gathers(0)

        def outer(io, carry):
            for p in (0, 1):
                i = 2 * io + p
                np_ = 1 - p

                @pl.when(i + 1 < nsb)
                def _pf():
                    fire_idx(i + 1, np_)

                wait_gathers(p)

                @pl.when(i + 1 < nsb)
                def _g():
                    wait_idx(i + 1, np_)
                    sub_rows(np_)
                    fire_gathers(np_)

                scale(p)
                do_scatter(p)
            return carry

        lax.fori_loop(0, nsb // 2, outer, 0)
        plsc.subcore_barrier()

        nwb_s = (nrows // rc - 1 - s) // NS + 1

        def wb(i, carry):
            off = (s + i * NS) * rc
            pltpu.sync_copy(acc.at[pl.ds(off, rc)], out.at[pl.ds(rbase + off, rc)])
            return carry

        lax.fori_loop(0, nwb_s, wb, 0)

    return prop


@functools.lru_cache(maxsize=None)
def _make_gather(nn, nb2):
    nw = NC * NS
    mesh = plsc.VectorSubcoreMesh(core_axis_name="c", subcore_axis_name="s")

    @functools.partial(
        pl.kernel,
        mesh=mesh,
        compiler_params=pltpu.CompilerParams(use_tc_tiling_on_sc=False),
        out_type=[
            jax.ShapeDtypeStruct((nb2 * KB, DIM), jnp.float32),
            jax.ShapeDtypeStruct((nb2 * KB, DIM), jnp.float32),
        ],
        scratch_types=[
            pltpu.VMEM((KB,), jnp.int32),
            pltpu.VMEM((KB, DIM), jnp.float32),
            pltpu.VMEM((KB, DIM), jnp.float32),
            pltpu.SemaphoreType.DMA,
        ],
    )
    def gath(e0, e1, e2, e3, idx, osum, o0, ib, buf0, bufa, sem):
        c = lax.axis_index("c")
        s = lax.axis_index("s")
        w = s * NC + c
        nblk = (nb2 - 1 - w) // nw + 1

        def blk(i, carry):
            b = w + i * nw
            base = b * KB
            pltpu.sync_copy(idx.at[pl.ds(base, KB)], ib)
            pltpu.async_copy(e0.at[ib], buf0, sem).wait()
            pltpu.sync_copy(buf0, o0.at[pl.ds(base, KB)])
            for tab in (e1, e2, e3):
                pltpu.async_copy(tab.at[ib], bufa, sem).wait()

                def add_(k, carry2):
                    buf0[k, pl.ds(0, LANES)] = (
                        buf0[k, pl.ds(0, LANES)] + bufa[k, pl.ds(0, LANES)]
                    )
                    buf0[k, pl.ds(LANES, LANES)] = (
                        buf0[k, pl.ds(LANES, LANES)] + bufa[k, pl.ds(LANES, LANES)]
                    )
                    return carry2

                lax.fori_loop(0, KB, add_, 0)
            pltpu.sync_copy(buf0, osum.at[pl.ds(base, KB)])
            return carry

        lax.fori_loop(0, nblk, blk, 0)

    return gath


def _loss_body(us_ref, ps_ref, ns_ref, u0_ref, p0_ref, n0_ref, reg_ref, out_ref, *, total):
    step = pl.program_id(0)

    @pl.when(step == 0)
    def _init():
        out_ref[0, 0] = 0.0

    us = us_ref[...] * 0.25
    ps = ps_ref[...] * 0.25
    ns = ns_ref[...] * 0.25
    pos_s = jnp.sum(us * ps, axis=1)
    neg_s = jnp.sum(us * ns, axis=1)
    x = pos_s - neg_s
    logsig = jnp.minimum(x, 0.0) - jnp.log(1.0 + jnp.exp(-jnp.abs(x)))
    r = reg_ref[0, 0]
    sq = (
        jnp.sum(u0_ref[...] * u0_ref[...])
        + jnp.sum(p0_ref[...] * p0_ref[...])
        + jnp.sum(n0_ref[...] * n0_ref[...])
    )
    part = -jnp.sum(logsig) / total + jnp.where(r > 0, r * sq / total, 0.0)
    out_ref[0, 0] += part


def _loss_tc(usum, psum, nsum, u0, p0, n0, regf):
    b = usum.shape[0]
    blk = 1024 if b % 1024 == 0 else b
    grid = b // blk
    row_spec = pl.BlockSpec((blk, DIM), lambda i: (i, 0))
    out = pl.pallas_call(
        functools.partial(_loss_body, total=float(b)),
        grid=(grid,),
        in_specs=[row_spec] * 6 + [pl.BlockSpec(memory_space=pltpu.SMEM)],
        out_specs=pl.BlockSpec(memory_space=pltpu.SMEM),
        out_shape=jax.ShapeDtypeStruct((1, 1), jnp.float32),
    )(usum, psum, nsum, u0, p0, n0, regf)
    return out[0, 0]


def kernel(users, pos, neg, reg, user_emb, item_emb, adj_rows, adj_cols, adj_vals):
    nu = user_emb.shape[0]
    ni = item_emb.shape[0]
    nn = nu + ni
    nnz = adj_rows.shape[0]
    half = nnz // 2
    nsb = -(-half // (NS * SB))
    nsb += nsb % 2
    half_pad = nsb * NS * SB

    rows_p = jnp.concatenate(
        [_pad1d(adj_rows[:half], half_pad, 0), _pad1d(adj_rows[half:], half_pad, nu)]
    )
    cols_p = jnp.concatenate(
        [_pad1d(adj_cols[:half], half_pad, 0), _pad1d(adj_cols[half:], half_pad, 0)]
    )
    vals_p = jnp.concatenate(
        [_pad1d(adj_vals[:half], half_pad, 0.0), _pad1d(adj_vals[half:], half_pad, 0.0)]
    )

    e0 = jnp.concatenate([user_emb, item_emb], axis=0)
    prop = _make_propagate(nn, nu, nsb)
    e1 = prop(e0, rows_p, cols_p, vals_p)
    e2 = prop(e1, rows_p, cols_p, vals_p)
    e3 = prop(e2, rows_p, cols_p, vals_p)

    b = users.shape[0]
    idx = jnp.concatenate(
        [users.astype(jnp.int32), pos.astype(jnp.int32) + nu, neg.astype(jnp.int32) + nu]
    )
    tot = 3 * b
    nb2 = -(-tot // KB)
    idx_p = _pad1d(idx, nb2 * KB, 0)
    gath = _make_gather(nn, nb2)
    gsum, g0 = gath(e0, e1, e2, e3, idx_p)

    regf = jnp.asarray(reg, jnp.float32).reshape(1, 1)
    return _loss_tc(
        gsum[:b], gsum[b : 2 * b], gsum[2 * b : tot],
        g0[:b], g0[b : 2 * b], g0[2 * b : tot],
        regf,
    )
